# baseline (speedup 1.0000x reference)
import jax
import jax.numpy as jnp
from jax.experimental import pallas as pl
from jax.experimental.pallas import tpu as pltpu


def _scores_body(a_ref, b_ref, c_ref, o_ref):
    s = jnp.maximum(jnp.maximum(a_ref[...], b_ref[...]), c_ref[...])
    o_ref[...] = jax.nn.sigmoid(s)


def kernel(points, cls_features):
    B, N, _ = cls_features.shape
    a = cls_features[:, :, 0]
    b = cls_features[:, :, 1]
    c = cls_features[:, :, 2]
    scores = pl.pallas_call(
        _scores_body,
        out_shape=jax.ShapeDtypeStruct((B, N), jnp.float32),
    )(a, b, c)
    _, idx = jax.lax.top_k(scores, 4096)
    return idx


# speedup vs baseline: 1.7504x; 1.7504x over previous
import functools

import jax
import jax.numpy as jnp
from jax import lax
from jax.experimental import pallas as pl
from jax.experimental.pallas import tpu as pltpu
from jax.experimental.pallas import tpu_sc as plsc

B = 16
N = 65536
K = 4096
CAP = 6144
NH = N // 2
CHUNK_PTS = 4096
CHUNK_W = CHUNK_PTS * 3
N_CHUNKS = NH // CHUNK_PTS
GPC = CHUNK_PTS // 16

_PAD_X = -1e30
_PAD_I = 0x7FFFFFFF


def _select_body(cls_ref, cx_ref, ci_ref,
                 stage, keys, hist, tot, tot2a, tot2b, candx, candi,
                 pstx, psti, sh_hist, sh_cx, sh_ci):
    ci = lax.axis_index("c")
    si = lax.axis_index("s")
    bl = si // 2
    h = si % 2
    b = ci * 8 + bl
    lane = lax.iota(jnp.int32, 16)
    ones = jnp.ones((16,), jnp.int32)
    base_pt = h * NH

    def zero_hist(i, _):
        hist[pl.ds(i * 16, 16)] = jnp.zeros((16,), jnp.int32)
        return 0

    lax.fori_loop(0, 256, zero_hist, 0)

    def chunk_body(cidx, _):
        pltpu.sync_copy(
            cls_ref.at[b, pl.ds(base_pt * 3 + cidx * CHUNK_W, CHUNK_W)], stage)

        def grp(g, _):
            i3 = (g * 16 + lane) * 3
            v0 = plsc.load_gather(stage, [i3])
            v1 = plsc.load_gather(stage, [i3 + 1])
            v2 = plsc.load_gather(stage, [i3 + 2])
            x = jnp.maximum(jnp.maximum(v0, v1), v2)
            u = plsc.bitcast(x, jnp.uint32)
            flip = jnp.where(x < 0.0,
                             jnp.full((16,), 0xFFFFFFFF, jnp.uint32),
                             jnp.full((16,), 0x80000000, jnp.uint32))
            key = u ^ flip
            keys[pl.ds(cidx * CHUNK_PTS + g * 16, 16)] = key
            d = lax.convert_element_type(key >> 24, jnp.int32)
            plsc.addupdate_scatter(hist, [d * 16 + lane], ones)
            return 0

        lax.fori_loop(0, GPC, grp, 0)
        return 0

    lax.fori_loop(0, N_CHUNKS, chunk_body, 0)

    def tsum(i, _):
        s = jnp.sum(hist[pl.ds(i * 16, 16)])
        plsc.store_scatter(tot, [jnp.broadcast_to(i, (16,))],
                           jnp.broadcast_to(s, (16,)), mask=lane == 0)
        return 0

    lax.fori_loop(0, 256, tsum, 0)
    pltpu.sync_copy(tot, sh_hist.at[bl, h])
    plsc.subcore_barrier()
    pltpu.sync_copy(sh_hist.at[bl, 0], tot2a)
    pltpu.sync_copy(sh_hist.at[bl, 1], tot2b)

    def decide(kneed):
        def dec(i, carry):
            run, bsel, cab, done = carry
            v = 15 - i
            hv = tot2a[pl.ds(v * 16, 16)] + tot2b[pl.ds(v * 16, 16)]
            r = lax.rev(hv, (0,))
            csum = plsc.cumsum(r) + run
            m = csum >= kneed
            i0 = jnp.min(jnp.where(m, lane, 16))
            sel = lane == i0
            csum_i0 = jnp.sum(jnp.where(sel, csum, 0))
            r_i0 = jnp.sum(jnp.where(sel, r, 0))
            blocktot = jnp.sum(hv)
            hit = jnp.logical_and(jnp.logical_not(done), i0 < 16)
            bsel = jnp.where(hit, v * 16 + 15 - i0, bsel)
            cab = jnp.where(hit, csum_i0 - r_i0, cab)
            done = jnp.logical_or(done, hit)
            return (run + blocktot, bsel, cab, done)

        _, bsel, cab, _ = lax.fori_loop(
            0, 16, dec, (jnp.int32(0), jnp.int32(0), jnp.int32(0), False))
        return bsel, cab

    b1, cab1 = decide(jnp.int32(K))
    k1 = K - cab1
    b1u = lax.convert_element_type(b1, jnp.uint32)

    lax.fori_loop(0, 256, zero_hist, 0)

    def l2(g, _):
        kv = keys[pl.ds(g * 16, 16)]
        m = (kv >> 24) == b1u
        d2 = lax.convert_element_type((kv >> 16) & jnp.uint32(0xFF), jnp.int32)
        plsc.addupdate_scatter(hist, [d2 * 16 + lane], ones, mask=m)
        return 0

    lax.fori_loop(0, NH // 16, l2, 0)
    lax.fori_loop(0, 256, tsum, 0)
    pltpu.sync_copy(tot, sh_hist.at[bl, h])
    plsc.subcore_barrier()
    pltpu.sync_copy(sh_hist.at[bl, 0], tot2a)
    pltpu.sync_copy(sh_hist.at[bl, 1], tot2b)
    b2, _ = decide(k1)
    b2u = lax.convert_element_type(b2, jnp.uint32)
    thr = (b1u << 24) | (b2u << 16)

    def prefill(i, _):
        candx[pl.ds(i * 16, 16)] = jnp.full((16,), _PAD_X, jnp.float32)
        candi[pl.ds(i * 16, 16)] = jnp.full((16,), _PAD_I, jnp.int32)
        return 0

    lax.fori_loop(0, CAP // 16, prefill, 0)

    def cp(g, off):
        kv = keys[pl.ds(g * 16, 16)]
        m = kv >= thr
        mi = lax.convert_element_type(m, jnp.int32)
        dest = jnp.minimum(off + plsc.cumsum(mi) - 1, CAP - 1)
        pos = kv >= jnp.uint32(0x80000000)
        u = kv ^ jnp.where(pos,
                           jnp.full((16,), 0x80000000, jnp.uint32),
                           jnp.full((16,), 0xFFFFFFFF, jnp.uint32))
        x = plsc.bitcast(u, jnp.float32)
        idxv = base_pt + g * 16 + lane
        plsc.store_scatter(candx, [dest], x, mask=m)
        plsc.store_scatter(candi, [dest], idxv, mask=m)
        return off + plsc.all_reduce_population_count(m)

    off = lax.fori_loop(0, NH // 16, cp, jnp.zeros((16,), jnp.int32))

    @pl.when(h == 1)
    def _():
        pltpu.sync_copy(candx, sh_cx.at[bl])
        pltpu.sync_copy(candi, sh_ci.at[bl])

    plsc.subcore_barrier()

    @pl.when(h == 0)
    def _():
        pltpu.sync_copy(sh_cx.at[bl], pstx)
        pltpu.sync_copy(sh_ci.at[bl], psti)

        def mg(j, _):
            xv = pstx[pl.ds(j * 16, 16)]
            iv = psti[pl.ds(j * 16, 16)]
            dest = jnp.minimum(off + j * 16 + lane, CAP - 1)
            plsc.store_scatter(candx, [dest], xv)
            plsc.store_scatter(candi, [dest], iv)
            return 0

        lax.fori_loop(0, CAP // 16, mg, 0)
        pltpu.sync_copy(candx, cx_ref.at[b])
        pltpu.sync_copy(candi, ci_ref.at[b])


@functools.partial(
    pl.kernel,
    out_type=[
        jax.ShapeDtypeStruct((B, CAP), jnp.float32),
        jax.ShapeDtypeStruct((B, CAP), jnp.int32),
    ],
    mesh=plsc.VectorSubcoreMesh(core_axis_name="c", subcore_axis_name="s"),
    compiler_params=pltpu.CompilerParams(
        use_tc_tiling_on_sc=False, needs_layout_passes=False),
    scratch_types=[
        pltpu.VMEM((CHUNK_W,), jnp.float32),
        pltpu.VMEM((NH,), jnp.uint32),
        pltpu.VMEM((4096,), jnp.int32),
        pltpu.VMEM((256,), jnp.int32),
        pltpu.VMEM((256,), jnp.int32),
        pltpu.VMEM((256,), jnp.int32),
        pltpu.VMEM((CAP,), jnp.float32),
        pltpu.VMEM((CAP,), jnp.int32),
        pltpu.VMEM((CAP,), jnp.float32),
        pltpu.VMEM((CAP,), jnp.int32),
        pltpu.VMEM_SHARED((8, 2, 256), jnp.int32),
        pltpu.VMEM_SHARED((8, CAP), jnp.float32),
        pltpu.VMEM_SHARED((8, CAP), jnp.int32),
    ],
)
def _select(cls_ref, cx_ref, ci_ref, *scratch):
    _select_body(cls_ref, cx_ref, ci_ref, *scratch)


def _sig_body(x_ref, o_ref):
    o_ref[...] = jax.nn.sigmoid(x_ref[...])


def _sort_body(sig_ref, cidx_ref, out_ref,
               sigv, keya, ida, keyb, idb, hist, baseb, scr_d, scr_k, scr_i):
    ci = lax.axis_index("c")
    si = lax.axis_index("s")
    bl = si // 2
    h = si % 2
    b = ci * 8 + bl
    lane = lax.iota(jnp.int32, 16)
    ones = jnp.ones((16,), jnp.int32)

    @pl.when(h == 0)
    def _():
        pltpu.sync_copy(sig_ref.at[b], sigv)
        pltpu.sync_copy(cidx_ref.at[b], ida)

        def mk(g, _):
            s = sigv[pl.ds(g * 16, 16)]
            u = plsc.bitcast(s, jnp.int32)
            keya[pl.ds(g * 16, 16)] = u ^ jnp.full((16,), -1, jnp.int32)
            return 0

        lax.fori_loop(0, CAP // 16, mk, 0)

        def radix_pass(shift, src_k, src_i, dst_k, dst_i):
            def z(i, _):
                hist[pl.ds(i * 16, 16)] = jnp.zeros((16,), jnp.int32)
                return 0

            lax.fori_loop(0, 256, z, 0)

            def hp(g, _):
                kv = src_k[pl.ds(g * 16, 16)]
                d = (kv >> shift) & 0xFF
                plsc.addupdate_scatter(hist, [d * 16 + lane], ones)
                return 0

            lax.fori_loop(0, CAP // 16, hp, 0)

            def sc(i, run):
                s = jnp.sum(hist[pl.ds(i * 16, 16)])
                plsc.store_scatter(baseb, [jnp.broadcast_to(i, (16,))],
                                   jnp.broadcast_to(run, (16,)),
                                   mask=lane == 0)
                return run + s

            lax.fori_loop(0, 256, sc, jnp.int32(0))

            def pp(g, _):
                kv = src_k[pl.ds(g * 16, 16)]
                iv = src_i[pl.ds(g * 16, 16)]
                d = (kv >> shift) & 0xFF
                sk = d * 16 + lane
                sks, lo = plsc.sort_key_val(sk, lane)
                ds_ = sks >> 4
                scr_d[...] = ds_
                prev = plsc.load_gather(scr_d, [jnp.maximum(lane - 1, 0)])
                isnew = jnp.logical_or(ds_ != prev, lane == 0)
                first = plsc.cummax(jnp.where(isnew, lane, 0))
                occ = lane - first
                basev = plsc.load_gather(baseb, [ds_])
                dest = basev + occ
                nxt = plsc.load_gather(scr_d, [jnp.minimum(lane + 1, 15)])
                islast = jnp.logical_or(ds_ != nxt, lane == 15)
                plsc.addupdate_scatter(baseb, [ds_], occ + 1, mask=islast)
                scr_k[...] = kv
                kvs = plsc.load_gather(scr_k, [lo])
                scr_i[...] = iv
                ivs = plsc.load_gather(scr_i, [lo])
                plsc.store_scatter(dst_k, [dest], kvs)
                plsc.store_scatter(dst_i, [dest], ivs)
                return 0

            lax.fori_loop(0, CAP // 16, pp, 0)

        radix_pass(0, keya, ida, keyb, idb)
        radix_pass(8, keyb, idb, keya, ida)
        radix_pass(16, keya, ida, keyb, idb)
        radix_pass(24, keyb, idb, keya, ida)

        pltpu.sync_copy(ida.at[pl.ds(0, K)], out_ref.at[b])


@functools.partial(
    pl.kernel,
    out_type=jax.ShapeDtypeStruct((B, K), jnp.int32),
    mesh=plsc.VectorSubcoreMesh(core_axis_name="c", subcore_axis_name="s"),
    compiler_params=pltpu.CompilerParams(
        use_tc_tiling_on_sc=False, needs_layout_passes=False),
    scratch_types=[
        pltpu.VMEM((CAP,), jnp.float32),
        pltpu.VMEM((CAP,), jnp.int32),
        pltpu.VMEM((CAP,), jnp.int32),
        pltpu.VMEM((CAP,), jnp.int32),
        pltpu.VMEM((CAP,), jnp.int32),
        pltpu.VMEM((4096,), jnp.int32),
        pltpu.VMEM((256,), jnp.int32),
        pltpu.VMEM((16,), jnp.int32),
        pltpu.VMEM((16,), jnp.int32),
        pltpu.VMEM((16,), jnp.int32),
    ],
)
def _sortk(sig_ref, cidx_ref, out_ref, *scratch):
    _sort_body(sig_ref, cidx_ref, out_ref, *scratch)


def kernel(points, cls_features):
    flat = cls_features.reshape(B, N * 3)
    cx, cidx = _select(flat)
    sig = pl.pallas_call(
        _sig_body,
        out_shape=jax.ShapeDtypeStruct((B, CAP), jnp.float32),
    )(cx)
    return _sortk(sig, cidx)


# speedup vs baseline: 4.2036x; 2.4015x over previous
import functools

import jax
import jax.numpy as jnp
from jax import lax
from jax.experimental import pallas as pl
from jax.experimental.pallas import tpu as pltpu
from jax.experimental.pallas import tpu_sc as plsc

B = 16
N = 65536
K = 4096
CAP = 6144
NH = N // 2

_PAD_I = 0x7FFFFFFF
_HALF_BITS = 0x3F000000


def _scores_body(a_ref, b_ref, c_ref, o_ref):
    s = jnp.maximum(jnp.maximum(a_ref[...], b_ref[...]), c_ref[...])
    o_ref[...] = lax.bitcast_convert_type(jax.nn.sigmoid(s), jnp.int32)


def _topk_body(keys_ref, out_ref,
               keys, hist, tot, tot2a, tot2b, keya, ida, keyb, idb,
               pstk, psti, scr_d, scr_k, scr_i, sh_hist, sh_ck, sh_ci):
    ci = lax.axis_index("c")
    si = lax.axis_index("s")
    bl = si // 2
    h = si % 2
    b = ci * 8 + bl
    lane = lax.iota(jnp.int32, 16)
    ones = jnp.ones((16,), jnp.int32)

    pltpu.sync_copy(keys_ref.at[b, pl.ds(h * NH, NH)], keys)

    def zero_hist(i, _):
        hist[pl.ds(i * 16, 16)] = jnp.zeros((16,), jnp.int32)
        return 0

    lax.fori_loop(0, 256, zero_hist, 0)

    def l1(g, _):
        kv = keys[pl.ds(g * 16, 16)]
        d = (kv >> 24) & 0xFF
        plsc.addupdate_scatter(hist, [d * 16 + lane], ones)
        return 0

    lax.fori_loop(0, NH // 16, l1, 0)

    def tsum(i, _):
        s = jnp.sum(hist[pl.ds(i * 16, 16)])
        plsc.store_scatter(tot, [jnp.broadcast_to(i, (16,))],
                           jnp.broadcast_to(s, (16,)), mask=lane == 0)
        return 0

    lax.fori_loop(0, 256, tsum, 0)
    pltpu.sync_copy(tot, sh_hist.at[bl, h])
    plsc.subcore_barrier()
    pltpu.sync_copy(sh_hist.at[bl, 0], tot2a)
    pltpu.sync_copy(sh_hist.at[bl, 1], tot2b)

    def decide(kneed):
        def dec(i, carry):
            run, bsel, cab, done = carry
            v = 15 - i
            hv = tot2a[pl.ds(v * 16, 16)] + tot2b[pl.ds(v * 16, 16)]
            r = lax.rev(hv, (0,))
            csum = plsc.cumsum(r) + run
            m = csum >= kneed
            i0 = jnp.min(jnp.where(m, lane, 16))
            sel = lane == i0
            csum_i0 = jnp.sum(jnp.where(sel, csum, 0))
            r_i0 = jnp.sum(jnp.where(sel, r, 0))
            blocktot = jnp.sum(hv)
            hit = jnp.logical_and(jnp.logical_not(done), i0 < 16)
            bsel = jnp.where(hit, v * 16 + 15 - i0, bsel)
            cab = jnp.where(hit, csum_i0 - r_i0, cab)
            done = jnp.logical_or(done, hit)
            return (run + blocktot, bsel, cab, done)

        _, bsel, cab, _ = lax.fori_loop(
            0, 16, dec, (jnp.int32(0), jnp.int32(0), jnp.int32(0), False))
        return bsel, cab

    b1, cab1 = decide(jnp.int32(K))
    k1 = K - cab1

    lax.fori_loop(0, 256, zero_hist, 0)

    def l2(g, _):
        kv = keys[pl.ds(g * 16, 16)]
        m = ((kv >> 24) & 0xFF) == b1
        d2 = (kv >> 16) & 0xFF
        plsc.addupdate_scatter(hist, [d2 * 16 + lane], ones, mask=m)
        return 0

    lax.fori_loop(0, NH // 16, l2, 0)
    lax.fori_loop(0, 256, tsum, 0)
    pltpu.sync_copy(tot, sh_hist.at[bl, h])
    plsc.subcore_barrier()
    pltpu.sync_copy(sh_hist.at[bl, 0], tot2a)
    pltpu.sync_copy(sh_hist.at[bl, 1], tot2b)
    b2, _ = decide(k1)
    thr = (b1 << 24) | (b2 << 16)

    padk = jnp.where(thr >= _HALF_BITS, jnp.int32(_HALF_BITS), jnp.int32(0))
    padsort = ~padk

    def prefill(i, _):
        keya[pl.ds(i * 16, 16)] = jnp.broadcast_to(padsort, (16,))
        ida[pl.ds(i * 16, 16)] = jnp.full((16,), _PAD_I, jnp.int32)
        return 0

    lax.fori_loop(0, CAP // 16, prefill, 0)

    def cp(g, off):
        kv = keys[pl.ds(g * 16, 16)]
        m = kv >= thr
        mi = lax.convert_element_type(m, jnp.int32)
        dest = jnp.minimum(off + plsc.cumsum(mi) - 1, CAP - 1)
        idxv = h * NH + g * 16 + lane
        plsc.store_scatter(keya, [dest], ~kv, mask=m)
        plsc.store_scatter(ida, [dest], idxv, mask=m)
        return off + plsc.all_reduce_population_count(m)

    off = lax.fori_loop(0, NH // 16, cp, jnp.zeros((16,), jnp.int32))

    @pl.when(h == 1)
    def _():
        pltpu.sync_copy(keya, sh_ck.at[bl])
        pltpu.sync_copy(ida, sh_ci.at[bl])

    plsc.subcore_barrier()

    @pl.when(h == 0)
    def _():
        pltpu.sync_copy(sh_ck.at[bl], pstk)
        pltpu.sync_copy(sh_ci.at[bl], psti)

        def mg(j, _):
            kv = pstk[pl.ds(j * 16, 16)]
            iv = psti[pl.ds(j * 16, 16)]
            dest = jnp.minimum(off + j * 16 + lane, CAP - 1)
            plsc.store_scatter(keya, [dest], kv)
            plsc.store_scatter(ida, [dest], iv)
            return 0

        lax.fori_loop(0, CAP // 16, mg, 0)

        def radix_pass(shift, src_k, src_i, dst_k, dst_i):
            lax.fori_loop(0, 256, zero_hist, 0)

            def hp(g, _):
                kv = src_k[pl.ds(g * 16, 16)]
                d = (kv >> shift) & 0xFF
                plsc.addupdate_scatter(hist, [d * 16 + lane], ones)
                return 0

            lax.fori_loop(0, CAP // 16, hp, 0)

            def sc(i, run):
                s = jnp.sum(hist[pl.ds(i * 16, 16)])
                plsc.store_scatter(tot, [jnp.broadcast_to(i, (16,))],
                                   jnp.broadcast_to(run, (16,)),
                                   mask=lane == 0)
                return run + s

            lax.fori_loop(0, 256, sc, jnp.int32(0))

            def pp(g, _):
                kv = src_k[pl.ds(g * 16, 16)]
                iv = src_i[pl.ds(g * 16, 16)]
                d = (kv >> shift) & 0xFF
                sk = d * 16 + lane
                sks, lo = plsc.sort_key_val(sk, lane)
                ds_ = sks >> 4
                scr_d[...] = ds_
                prev = plsc.load_gather(scr_d, [jnp.maximum(lane - 1, 0)])
                isnew = jnp.logical_or(ds_ != prev, lane == 0)
                first = plsc.cummax(jnp.where(isnew, lane, 0))
                occ = lane - first
                basev = plsc.load_gather(tot, [ds_])
                dest = basev + occ
                nxt = plsc.load_gather(scr_d, [jnp.minimum(lane + 1, 15)])
                islast = jnp.logical_or(ds_ != nxt, lane == 15)
                plsc.addupdate_scatter(tot, [ds_], occ + 1, mask=islast)
                scr_k[...] = kv
                kvs = plsc.load_gather(scr_k, [lo])
                scr_i[...] = iv
                ivs = plsc.load_gather(scr_i, [lo])
                plsc.store_scatter(dst_k, [dest], kvs)
                plsc.store_scatter(dst_i, [dest], ivs)
                return 0

            lax.fori_loop(0, CAP // 16, pp, 0)

        radix_pass(0, keya, ida, keyb, idb)
        radix_pass(8, keyb, idb, keya, ida)
        radix_pass(16, keya, ida, keyb, idb)

        @pl.when(thr < _HALF_BITS)
        def _():
            radix_pass(24, keyb, idb, keya, ida)

            def cpy(j, _):
                idb[pl.ds(j * 16, 16)] = ida[pl.ds(j * 16, 16)]
                return 0

            lax.fori_loop(0, K // 16, cpy, 0)

        pltpu.sync_copy(idb.at[pl.ds(0, K)], out_ref.at[b])


@functools.partial(
    pl.kernel,
    out_type=jax.ShapeDtypeStruct((B, K), jnp.int32),
    mesh=plsc.VectorSubcoreMesh(core_axis_name="c", subcore_axis_name="s"),
    compiler_params=pltpu.CompilerParams(
        use_tc_tiling_on_sc=False, needs_layout_passes=False),
    scratch_types=[
        pltpu.VMEM((NH,), jnp.int32),
        pltpu.VMEM((4096,), jnp.int32),
        pltpu.VMEM((256,), jnp.int32),
        pltpu.VMEM((256,), jnp.int32),
        pltpu.VMEM((256,), jnp.int32),
        pltpu.VMEM((CAP,), jnp.int32),
        pltpu.VMEM((CAP,), jnp.int32),
        pltpu.VMEM((CAP,), jnp.int32),
        pltpu.VMEM((CAP,), jnp.int32),
        pltpu.VMEM((CAP,), jnp.int32),
        pltpu.VMEM((CAP,), jnp.int32),
        pltpu.VMEM((16,), jnp.int32),
        pltpu.VMEM((16,), jnp.int32),
        pltpu.VMEM((16,), jnp.int32),
        pltpu.VMEM_SHARED((8, 2, 256), jnp.int32),
        pltpu.VMEM_SHARED((8, CAP), jnp.int32),
        pltpu.VMEM_SHARED((8, CAP), jnp.int32),
    ],
)
def _sc_topk(keys_ref, out_ref, *scratch):
    _topk_body(keys_ref, out_ref, *scratch)


def kernel(points, cls_features):
    a = cls_features[:, :, 0]
    b = cls_features[:, :, 1]
    c = cls_features[:, :, 2]
    keys = pl.pallas_call(
        _scores_body,
        out_shape=jax.ShapeDtypeStruct((B, N), jnp.int32),
    )(a, b, c)
    return _sc_topk(keys)


# speedup vs baseline: 4.3855x; 1.0433x over previous
import functools

import jax
import jax.numpy as jnp
from jax import lax
from jax.experimental import pallas as pl
from jax.experimental.pallas import tpu as pltpu
from jax.experimental.pallas import tpu_sc as plsc

B = 16
N = 65536
K = 4096
CAP = 6144
NH = N // 2

_PAD_I = 0x7FFFFFFF
_HALF_BITS = 0x3F000000


def _scores_body(a_ref, b_ref, c_ref, o_ref):
    s = jnp.maximum(jnp.maximum(a_ref[...], b_ref[...]), c_ref[...])
    o_ref[...] = lax.bitcast_convert_type(jax.nn.sigmoid(s), jnp.int32)


def _topk_body(keys_ref, out_ref,
               keys, hist, tot, tot2a, tot2b, keya, ida, keyb, idb,
               pstk, psti, scr_d, scr_k, scr_i, sh_hist, sh_ck, sh_ci):
    ci = lax.axis_index("c")
    si = lax.axis_index("s")
    bl = si // 2
    h = si % 2
    b = ci * 8 + bl
    lane = lax.iota(jnp.int32, 16)
    ones = jnp.ones((16,), jnp.int32)

    lane256 = lane * 256

    pltpu.sync_copy(keys_ref.at[b, pl.ds(h * NH, NH)], keys)

    def zero_hist(i, _):
        hist[pl.ds(i * 16, 16)] = jnp.zeros((16,), jnp.int32)
        return 0

    lax.fori_loop(0, 256, zero_hist, 0)

    def l1(g, _):
        kv = keys[pl.ds(g * 16, 16)]
        d = (kv >> 24) & 0xFF
        plsc.addupdate_scatter(hist, [d + lane256], ones)
        return 0

    lax.fori_loop(0, NH // 16, l1, 0)

    def tsum(i, _):
        acc = hist[pl.ds(i * 16, 16)]
        for l in range(1, 16):
            acc = acc + hist[pl.ds(l * 256 + i * 16, 16)]
        tot[pl.ds(i * 16, 16)] = acc
        return 0

    lax.fori_loop(0, 16, tsum, 0)
    pltpu.sync_copy(tot, sh_hist.at[bl, h])
    plsc.subcore_barrier()
    pltpu.sync_copy(sh_hist.at[bl, 0], tot2a)
    pltpu.sync_copy(sh_hist.at[bl, 1], tot2b)

    def decide(kneed):
        def dec(i, carry):
            run, bsel, cab, done = carry
            v = 15 - i
            hv = tot2a[pl.ds(v * 16, 16)] + tot2b[pl.ds(v * 16, 16)]
            r = lax.rev(hv, (0,))
            csum = plsc.cumsum(r) + run
            m = csum >= kneed
            i0 = jnp.min(jnp.where(m, lane, 16))
            sel = lane == i0
            csum_i0 = jnp.sum(jnp.where(sel, csum, 0))
            r_i0 = jnp.sum(jnp.where(sel, r, 0))
            blocktot = jnp.sum(hv)
            hit = jnp.logical_and(jnp.logical_not(done), i0 < 16)
            bsel = jnp.where(hit, v * 16 + 15 - i0, bsel)
            cab = jnp.where(hit, csum_i0 - r_i0, cab)
            done = jnp.logical_or(done, hit)
            return (run + blocktot, bsel, cab, done)

        _, bsel, cab, _ = lax.fori_loop(
            0, 16, dec, (jnp.int32(0), jnp.int32(0), jnp.int32(0), False))
        return bsel, cab

    b1, cab1 = decide(jnp.int32(K))
    k1 = K - cab1

    lax.fori_loop(0, 256, zero_hist, 0)

    def l2(g, _):
        kv = keys[pl.ds(g * 16, 16)]
        m = ((kv >> 24) & 0xFF) == b1
        d2 = (kv >> 16) & 0xFF
        plsc.addupdate_scatter(hist, [d2 + lane256], ones, mask=m)
        return 0

    lax.fori_loop(0, NH // 16, l2, 0)
    lax.fori_loop(0, 16, tsum, 0)
    pltpu.sync_copy(tot, sh_hist.at[bl, h])
    plsc.subcore_barrier()
    pltpu.sync_copy(sh_hist.at[bl, 0], tot2a)
    pltpu.sync_copy(sh_hist.at[bl, 1], tot2b)
    b2, _ = decide(k1)
    thr = (b1 << 24) | (b2 << 16)

    padk = jnp.where(thr >= _HALF_BITS, jnp.int32(_HALF_BITS), jnp.int32(0))
    padsort = ~padk

    def prefill(i, _):
        keya[pl.ds(i * 16, 16)] = jnp.broadcast_to(padsort, (16,))
        ida[pl.ds(i * 16, 16)] = jnp.full((16,), _PAD_I, jnp.int32)
        return 0

    lax.fori_loop(0, CAP // 16, prefill, 0)

    def cp(g, off):
        kv = keys[pl.ds(g * 16, 16)]
        m = kv >= thr
        mi = lax.convert_element_type(m, jnp.int32)
        dest = jnp.minimum(off + plsc.cumsum(mi) - 1, CAP - 1)
        idxv = h * NH + g * 16 + lane
        plsc.store_scatter(keya, [dest], ~kv, mask=m)
        plsc.store_scatter(ida, [dest], idxv, mask=m)
        return off + plsc.all_reduce_population_count(m)

    off = lax.fori_loop(0, NH // 16, cp, jnp.zeros((16,), jnp.int32))

    @pl.when(h == 1)
    def _():
        pltpu.sync_copy(keya, sh_ck.at[bl])
        pltpu.sync_copy(ida, sh_ci.at[bl])
        scr_k[...] = off
        pltpu.sync_copy(scr_k, sh_hist.at[bl, 1, pl.ds(0, 16)])

    plsc.subcore_barrier()

    @pl.when(h == 0)
    def _():
        pltpu.sync_copy(sh_ck.at[bl], pstk)
        pltpu.sync_copy(sh_ci.at[bl], psti)
        pltpu.sync_copy(sh_hist.at[bl, 1, pl.ds(0, 16)], scr_k)
        n1 = jnp.max(scr_k[...])
        cn = jnp.max(off) + n1
        gn = jnp.minimum((cn + 15) >> 4, CAP // 16)

        def mg(j, _):
            kv = pstk[pl.ds(j * 16, 16)]
            iv = psti[pl.ds(j * 16, 16)]
            dest = jnp.minimum(off + j * 16 + lane, CAP - 1)
            plsc.store_scatter(keya, [dest], kv)
            plsc.store_scatter(ida, [dest], iv)
            return 0

        lax.fori_loop(0, CAP // 16, mg, 0)

        def radix_pass(shift, src_k, src_i, dst_k, dst_i):
            lax.fori_loop(0, 256, zero_hist, 0)

            def hp(g, _):
                kv = src_k[pl.ds(g * 16, 16)]
                d = (kv >> shift) & 0xFF
                plsc.addupdate_scatter(hist, [d + lane256], ones)
                return 0

            lax.fori_loop(0, gn, hp, 0)

            def sc(i, run):
                acc = hist[pl.ds(i * 16, 16)]
                for l in range(1, 16):
                    acc = acc + hist[pl.ds(l * 256 + i * 16, 16)]
                tot[pl.ds(i * 16, 16)] = plsc.cumsum(acc) - acc + run
                return run + jnp.sum(acc)

            lax.fori_loop(0, 16, sc, jnp.int32(0))

            def pp(g, _):
                kv = src_k[pl.ds(g * 16, 16)]
                iv = src_i[pl.ds(g * 16, 16)]
                d = (kv >> shift) & 0xFF
                sk = d * 16 + lane
                sks, lo = plsc.sort_key_val(sk, lane)
                ds_ = sks >> 4
                scr_d[...] = ds_
                prev = plsc.load_gather(scr_d, [jnp.maximum(lane - 1, 0)])
                isnew = jnp.logical_or(ds_ != prev, lane == 0)
                first = plsc.cummax(jnp.where(isnew, lane, 0))
                occ = lane - first
                basev = plsc.load_gather(tot, [ds_])
                dest = basev + occ
                nxt = plsc.load_gather(scr_d, [jnp.minimum(lane + 1, 15)])
                islast = jnp.logical_or(ds_ != nxt, lane == 15)
                plsc.addupdate_scatter(tot, [ds_], occ + 1, mask=islast)
                scr_k[...] = kv
                kvs = plsc.load_gather(scr_k, [lo])
                scr_i[...] = iv
                ivs = plsc.load_gather(scr_i, [lo])
                plsc.store_scatter(dst_k, [dest], kvs)
                plsc.store_scatter(dst_i, [dest], ivs)
                return 0

            lax.fori_loop(0, gn, pp, 0)

        radix_pass(0, keya, ida, keyb, idb)
        radix_pass(8, keyb, idb, keya, ida)
        radix_pass(16, keya, ida, keyb, idb)

        @pl.when(thr < _HALF_BITS)
        def _():
            radix_pass(24, keyb, idb, keya, ida)

            def cpy(j, _):
                idb[pl.ds(j * 16, 16)] = ida[pl.ds(j * 16, 16)]
                return 0

            lax.fori_loop(0, K // 16, cpy, 0)

        pltpu.sync_copy(idb.at[pl.ds(0, K)], out_ref.at[b])


@functools.partial(
    pl.kernel,
    out_type=jax.ShapeDtypeStruct((B, K), jnp.int32),
    mesh=plsc.VectorSubcoreMesh(core_axis_name="c", subcore_axis_name="s"),
    compiler_params=pltpu.CompilerParams(
        use_tc_tiling_on_sc=False, needs_layout_passes=False),
    scratch_types=[
        pltpu.VMEM((NH,), jnp.int32),
        pltpu.VMEM((4096,), jnp.int32),
        pltpu.VMEM((256,), jnp.int32),
        pltpu.VMEM((256,), jnp.int32),
        pltpu.VMEM((256,), jnp.int32),
        pltpu.VMEM((CAP,), jnp.int32),
        pltpu.VMEM((CAP,), jnp.int32),
        pltpu.VMEM((CAP,), jnp.int32),
        pltpu.VMEM((CAP,), jnp.int32),
        pltpu.VMEM((CAP,), jnp.int32),
        pltpu.VMEM((CAP,), jnp.int32),
        pltpu.VMEM((16,), jnp.int32),
        pltpu.VMEM((16,), jnp.int32),
        pltpu.VMEM((16,), jnp.int32),
        pltpu.VMEM_SHARED((8, 2, 256), jnp.int32),
        pltpu.VMEM_SHARED((8, CAP), jnp.int32),
        pltpu.VMEM_SHARED((8, CAP), jnp.int32),
    ],
)
def _sc_topk(keys_ref, out_ref, *scratch):
    _topk_body(keys_ref, out_ref, *scratch)


def kernel(points, cls_features):
    a = cls_features[:, :, 0]
    b = cls_features[:, :, 1]
    c = cls_features[:, :, 2]
    keys = pl.pallas_call(
        _scores_body,
        out_shape=jax.ShapeDtypeStruct((B, N), jnp.int32),
    )(a, b, c)
    return _sc_topk(keys)


# speedup vs baseline: 4.4561x; 1.0161x over previous
import functools

import jax
import jax.numpy as jnp
from jax import lax
from jax.experimental import pallas as pl
from jax.experimental.pallas import tpu as pltpu
from jax.experimental.pallas import tpu_sc as plsc

B = 16
N = 65536
K = 4096
CAP = 6144
NH = N // 2

_PAD_I = 0x7FFFFFFF
_HALF_BITS = 0x3F000000


def _scores_body(a_ref, b_ref, c_ref, o_ref):
    s = jnp.maximum(jnp.maximum(a_ref[...], b_ref[...]), c_ref[...])
    o_ref[...] = lax.bitcast_convert_type(jax.nn.sigmoid(s), jnp.int32)


def _topk_body(keys_ref, out_ref,
               keys, hist, tot, tot2a, tot2b, keya, ida, keyb, idb,
               pstk, psti, scr_d, scr_k, scr_i, sh_hist, sh_ck, sh_ci):
    ci = lax.axis_index("c")
    si = lax.axis_index("s")
    bl = si // 2
    h = si % 2
    b = ci * 8 + bl
    lane = lax.iota(jnp.int32, 16)
    ones = jnp.ones((16,), jnp.int32)

    lane256 = lane * 256

    pltpu.sync_copy(keys_ref.at[b, pl.ds(h * NH, NH)], keys)

    def zero_hist(i, _):
        hist[pl.ds(i * 16, 16)] = jnp.zeros((16,), jnp.int32)
        return 0

    lax.fori_loop(0, 256, zero_hist, 0)

    def l1(g, _):
        kv = keys[pl.ds(g * 16, 16)]
        d = (kv >> 24) & 0xFF
        plsc.addupdate_scatter(hist, [d + lane256], ones)
        return 0

    lax.fori_loop(0, NH // 16, l1, 0, unroll=8)

    def tsum(i, _):
        acc = hist[pl.ds(i * 16, 16)]
        for l in range(1, 16):
            acc = acc + hist[pl.ds(l * 256 + i * 16, 16)]
        tot[pl.ds(i * 16, 16)] = acc
        return 0

    lax.fori_loop(0, 16, tsum, 0)
    pltpu.sync_copy(tot, sh_hist.at[bl, h])
    plsc.subcore_barrier()
    pltpu.sync_copy(sh_hist.at[bl, 0], tot2a)
    pltpu.sync_copy(sh_hist.at[bl, 1], tot2b)

    def decide(kneed):
        def dec(i, carry):
            run, bsel, cab, done = carry
            v = 15 - i
            hv = tot2a[pl.ds(v * 16, 16)] + tot2b[pl.ds(v * 16, 16)]
            r = lax.rev(hv, (0,))
            csum = plsc.cumsum(r) + run
            m = csum >= kneed
            i0 = jnp.min(jnp.where(m, lane, 16))
            sel = lane == i0
            csum_i0 = jnp.sum(jnp.where(sel, csum, 0))
            r_i0 = jnp.sum(jnp.where(sel, r, 0))
            blocktot = jnp.sum(hv)
            hit = jnp.logical_and(jnp.logical_not(done), i0 < 16)
            bsel = jnp.where(hit, v * 16 + 15 - i0, bsel)
            cab = jnp.where(hit, csum_i0 - r_i0, cab)
            done = jnp.logical_or(done, hit)
            return (run + blocktot, bsel, cab, done)

        _, bsel, cab, _ = lax.fori_loop(
            0, 16, dec, (jnp.int32(0), jnp.int32(0), jnp.int32(0), False))
        return bsel, cab

    b1, cab1 = decide(jnp.int32(K))
    k1 = K - cab1

    lax.fori_loop(0, 256, zero_hist, 0)

    def l2(g, _):
        kv = keys[pl.ds(g * 16, 16)]
        m = ((kv >> 24) & 0xFF) == b1
        d2 = (kv >> 16) & 0xFF
        plsc.addupdate_scatter(hist, [d2 + lane256], ones, mask=m)
        return 0

    lax.fori_loop(0, NH // 16, l2, 0, unroll=8)
    lax.fori_loop(0, 16, tsum, 0)
    pltpu.sync_copy(tot, sh_hist.at[bl, h])
    plsc.subcore_barrier()
    pltpu.sync_copy(sh_hist.at[bl, 0], tot2a)
    pltpu.sync_copy(sh_hist.at[bl, 1], tot2b)
    b2, _ = decide(k1)
    thr = (b1 << 24) | (b2 << 16)

    padk = jnp.where(thr >= _HALF_BITS, jnp.int32(_HALF_BITS), jnp.int32(0))
    padsort = ~padk

    def prefill(i, _):
        keya[pl.ds(i * 16, 16)] = jnp.broadcast_to(padsort, (16,))
        ida[pl.ds(i * 16, 16)] = jnp.full((16,), _PAD_I, jnp.int32)
        return 0

    lax.fori_loop(0, CAP // 16, prefill, 0, unroll=8)

    def cp(g, off):
        kv = keys[pl.ds(g * 16, 16)]
        m = kv >= thr
        mi = lax.convert_element_type(m, jnp.int32)
        dest = jnp.minimum(off + plsc.cumsum(mi) - 1, CAP - 1)
        idxv = h * NH + g * 16 + lane
        plsc.store_scatter(keya, [dest], ~kv, mask=m)
        plsc.store_scatter(ida, [dest], idxv, mask=m)
        return off + plsc.all_reduce_population_count(m)

    off = lax.fori_loop(0, NH // 16, cp, jnp.zeros((16,), jnp.int32), unroll=4)

    @pl.when(h == 1)
    def _():
        pltpu.sync_copy(keya, sh_ck.at[bl])
        pltpu.sync_copy(ida, sh_ci.at[bl])
        scr_k[...] = off
        pltpu.sync_copy(scr_k, sh_hist.at[bl, 1, pl.ds(0, 16)])

    plsc.subcore_barrier()

    @pl.when(h == 0)
    def _():
        pltpu.sync_copy(sh_ck.at[bl], pstk)
        pltpu.sync_copy(sh_ci.at[bl], psti)
        pltpu.sync_copy(sh_hist.at[bl, 1, pl.ds(0, 16)], scr_k)
        n1 = jnp.max(scr_k[...])
        cn = jnp.max(off) + n1
        gn = jnp.minimum((cn + 15) >> 4, CAP // 16)

        def mg(j, _):
            kv = pstk[pl.ds(j * 16, 16)]
            iv = psti[pl.ds(j * 16, 16)]
            dest = jnp.minimum(off + j * 16 + lane, CAP - 1)
            plsc.store_scatter(keya, [dest], kv)
            plsc.store_scatter(ida, [dest], iv)
            return 0

        lax.fori_loop(0, CAP // 16, mg, 0, unroll=4)

        def radix_pass(shift, src_k, src_i, dst_k, dst_i):
            lax.fori_loop(0, 256, zero_hist, 0)

            def hp(g, _):
                kv = src_k[pl.ds(g * 16, 16)]
                d = (kv >> shift) & 0xFF
                plsc.addupdate_scatter(hist, [d + lane256], ones)
                return 0

            lax.fori_loop(0, gn, hp, 0)

            def sc(i, run):
                acc = hist[pl.ds(i * 16, 16)]
                for l in range(1, 16):
                    acc = acc + hist[pl.ds(l * 256 + i * 16, 16)]
                tot[pl.ds(i * 16, 16)] = plsc.cumsum(acc) - acc + run
                return run + jnp.sum(acc)

            lax.fori_loop(0, 16, sc, jnp.int32(0))

            def pp(g, _):
                kv = src_k[pl.ds(g * 16, 16)]
                iv = src_i[pl.ds(g * 16, 16)]
                d = (kv >> shift) & 0xFF
                sk = d * 16 + lane
                sks, lo = plsc.sort_key_val(sk, lane)
                ds_ = sks >> 4
                scr_d[...] = ds_
                prev = plsc.load_gather(scr_d, [jnp.maximum(lane - 1, 0)])
                isnew = jnp.logical_or(ds_ != prev, lane == 0)
                first = plsc.cummax(jnp.where(isnew, lane, 0))
                occ = lane - first
                basev = plsc.load_gather(tot, [ds_])
                dest = basev + occ
                nxt = plsc.load_gather(scr_d, [jnp.minimum(lane + 1, 15)])
                islast = jnp.logical_or(ds_ != nxt, lane == 15)
                plsc.addupdate_scatter(tot, [ds_], occ + 1, mask=islast)
                scr_k[...] = kv
                kvs = plsc.load_gather(scr_k, [lo])
                scr_i[...] = iv
                ivs = plsc.load_gather(scr_i, [lo])
                plsc.store_scatter(dst_k, [dest], kvs)
                plsc.store_scatter(dst_i, [dest], ivs)
                return 0

            lax.fori_loop(0, gn, pp, 0)

        radix_pass(0, keya, ida, keyb, idb)
        radix_pass(8, keyb, idb, keya, ida)
        radix_pass(16, keya, ida, keyb, idb)

        @pl.when(thr < _HALF_BITS)
        def _():
            radix_pass(24, keyb, idb, keya, ida)

            def cpy(j, _):
                idb[pl.ds(j * 16, 16)] = ida[pl.ds(j * 16, 16)]
                return 0

            lax.fori_loop(0, K // 16, cpy, 0)

        pltpu.sync_copy(idb.at[pl.ds(0, K)], out_ref.at[b])


@functools.partial(
    pl.kernel,
    out_type=jax.ShapeDtypeStruct((B, K), jnp.int32),
    mesh=plsc.VectorSubcoreMesh(core_axis_name="c", subcore_axis_name="s"),
    compiler_params=pltpu.CompilerParams(
        use_tc_tiling_on_sc=False, needs_layout_passes=False),
    scratch_types=[
        pltpu.VMEM((NH,), jnp.int32),
        pltpu.VMEM((4096,), jnp.int32),
        pltpu.VMEM((256,), jnp.int32),
        pltpu.VMEM((256,), jnp.int32),
        pltpu.VMEM((256,), jnp.int32),
        pltpu.VMEM((CAP,), jnp.int32),
        pltpu.VMEM((CAP,), jnp.int32),
        pltpu.VMEM((CAP,), jnp.int32),
        pltpu.VMEM((CAP,), jnp.int32),
        pltpu.VMEM((CAP,), jnp.int32),
        pltpu.VMEM((CAP,), jnp.int32),
        pltpu.VMEM((16,), jnp.int32),
        pltpu.VMEM((16,), jnp.int32),
        pltpu.VMEM((16,), jnp.int32),
        pltpu.VMEM_SHARED((8, 2, 256), jnp.int32),
        pltpu.VMEM_SHARED((8, CAP), jnp.int32),
        pltpu.VMEM_SHARED((8, CAP), jnp.int32),
    ],
)
def _sc_topk(keys_ref, out_ref, *scratch):
    _topk_body(keys_ref, out_ref, *scratch)


def kernel(points, cls_features):
    a = cls_features[:, :, 0]
    b = cls_features[:, :, 1]
    c = cls_features[:, :, 2]
    keys = pl.pallas_call(
        _scores_body,
        out_shape=jax.ShapeDtypeStruct((B, N), jnp.int32),
    )(a, b, c)
    return _sc_topk(keys)
